# baseline (speedup 1.0000x reference)
import functools

import jax
import jax.numpy as jnp
from jax import lax
from jax.experimental import pallas as pl
from jax.experimental.pallas import tpu as pltpu
from jax.experimental.pallas import tpu_sc as plsc

_N = 10000
_B = 2048
_DIN = 128
_W = 1280
_PLANES = 8
_R = 4224

_NC, _NS = 2, 16
_NW = _NC * _NS
_BPW = _B // _NW

_BB = 256
_GRID = _B // _BB


@functools.cache
def _make_sc_gather():
    @functools.partial(
        pl.kernel,
        mesh=plsc.VectorSubcoreMesh(core_axis_name="c", subcore_axis_name="s"),
        out_type=(
            jax.ShapeDtypeStruct((_B, _W), jnp.int32),
            jax.ShapeDtypeStruct((_B, _W), jnp.int32),
            jax.ShapeDtypeStruct((_B, _DIN), jnp.float32),
            jax.ShapeDtypeStruct((_B, _DIN), jnp.float32),
        ),
        scratch_types=[
            pltpu.VMEM((_BPW,), jnp.int32),
            pltpu.VMEM((_BPW, _W), jnp.int32),
            pltpu.VMEM((_BPW, _DIN), jnp.float32),
            pltpu.SemaphoreType.DMA,
        ],
    )
    def _sc_gather(nib_hbm, x_hbm, ci_hbm, cj_hbm, ii_hbm, jj_hbm,
                   ni_out, nj_out, xi_out, xj_out,
                   idx_v, rowsn_v, rowsx_v, sem):
        wid = lax.axis_index("s") * _NC + lax.axis_index("c")
        base = wid * _BPW
        pltpu.sync_copy(ci_hbm.at[pl.ds(base, _BPW)], idx_v)
        pltpu.async_copy(nib_hbm.at[idx_v], rowsn_v, sem).wait()
        pltpu.sync_copy(rowsn_v, ni_out.at[pl.ds(base, _BPW)])
        pltpu.sync_copy(ii_hbm.at[pl.ds(base, _BPW)], idx_v)
        pltpu.async_copy(x_hbm.at[idx_v], rowsx_v, sem).wait()
        pltpu.sync_copy(rowsx_v, xi_out.at[pl.ds(base, _BPW)])
        pltpu.sync_copy(cj_hbm.at[pl.ds(base, _BPW)], idx_v)
        pltpu.async_copy(nib_hbm.at[idx_v], rowsn_v, sem).wait()
        pltpu.sync_copy(rowsn_v, nj_out.at[pl.ds(base, _BPW)])
        pltpu.sync_copy(jj_hbm.at[pl.ds(base, _BPW)], idx_v)
        pltpu.async_copy(x_hbm.at[idx_v], rowsx_v, sem).wait()
        pltpu.sync_copy(rowsx_v, xj_out.at[pl.ds(base, _BPW)])

    return _sc_gather


def _tc_body(ni_ref, nj_ref, xi_ref, xj_ref, xt_ref,
             xcn_w1_ref, xcn_b1_ref, xcn_w2_ref, xcn_b2_ref,
             xij_w1_ref, xij_b1_ref, xij_w2_ref, xij_b2_ref,
             lin0_w_ref, lin0_b_ref, lin1_w_ref, lin1_b_ref,
             lin2_w_ref, lin2_b_ref, out_ref):
    f32 = jnp.float32
    ni = ni_ref[...]
    nj = nj_ref[...]
    acc = jnp.zeros((_BB, _DIN), f32)
    for p in range(_PLANES):
        s = 4 * p
        mi = (jnp.right_shift(ni, s) & 15) > 0
        mj = (jnp.right_shift(nj, s) & 15) > 0
        m = (mi & mj).astype(f32)
        acc = acc + jnp.dot(m, xt_ref[p], preferred_element_type=f32)
    relu = lambda a: jnp.maximum(a, 0.0)
    xcn = relu(jnp.dot(acc, xcn_w1_ref[...], preferred_element_type=f32) + xcn_b1_ref[...])
    xcn = jnp.dot(xcn, xcn_w2_ref[...], preferred_element_type=f32) + xcn_b2_ref[...]
    xij = xi_ref[...] * xj_ref[...]
    xij = relu(jnp.dot(xij, xij_w1_ref[...], preferred_element_type=f32) + xij_b1_ref[...])
    xij = jnp.dot(xij, xij_w2_ref[...], preferred_element_type=f32) + xij_b2_ref[...]
    h = xcn + xij
    h = relu(jnp.dot(h, lin0_w_ref[...], preferred_element_type=f32) + lin0_b_ref[...])
    h = relu(jnp.dot(h, lin1_w_ref[...], preferred_element_type=f32) + lin1_b_ref[...])
    out_ref[...] = jnp.dot(h, lin2_w_ref[...], preferred_element_type=f32) + lin2_b_ref[...]


def _full(shape):
    return pl.BlockSpec(shape, lambda b: tuple(0 for _ in shape))


def kernel(x, adj_t, tar_ei, xcn_w1, xcn_b1, xcn_w2, xcn_b2,
           xij_w1, xij_b1, xij_w2, xij_b2,
           lin0_w, lin0_b, lin1_w, lin1_b, lin2_w, lin2_b):
    f32 = jnp.float32
    x = x.astype(f32)

    u = adj_t[0].astype(jnp.int32)
    v = adj_t[1].astype(jnp.int32)
    ii = tar_ei[0].astype(jnp.int32)
    jj = tar_ei[1].astype(jnp.int32)

    mask = jnp.zeros((_N,), jnp.int32).at[ii].set(1).at[jj].set(1)
    rowid = jnp.cumsum(mask) - mask
    ci = jnp.take(rowid, ii)
    cj = jnp.take(rowid, jj)

    erow = jnp.where(jnp.take(mask, u) > 0, jnp.take(rowid, u), _R - 1)
    widx = erow * _W + jnp.right_shift(v, 3)
    nibval = jnp.left_shift(jnp.int32(1), jnp.left_shift(v & 7, 2))
    nib = jnp.zeros((_R * _W,), jnp.int32).at[widx].add(nibval).reshape(_R, _W)

    xp = jnp.concatenate([x, jnp.zeros((_W * _PLANES - _N, _DIN), f32)], axis=0)
    xt = xp.reshape(_W, _PLANES, _DIN).transpose(1, 0, 2)

    ni, nj, xi, xj = _make_sc_gather()(nib, x, ci, cj, ii, jj)

    out = pl.pallas_call(
        _tc_body,
        grid=(_GRID,),
        in_specs=[
            pl.BlockSpec((_BB, _W), lambda b: (b, 0)),
            pl.BlockSpec((_BB, _W), lambda b: (b, 0)),
            pl.BlockSpec((_BB, _DIN), lambda b: (b, 0)),
            pl.BlockSpec((_BB, _DIN), lambda b: (b, 0)),
            _full((_PLANES, _W, _DIN)),
            _full((_DIN, 256)), _full((1, 256)),
            _full((256, 256)), _full((1, 256)),
            _full((_DIN, 256)), _full((1, 256)),
            _full((256, 256)), _full((1, 256)),
            _full((256, 256)), _full((1, 256)),
            _full((256, 256)), _full((1, 256)),
            _full((256, 1)), _full((1, 1)),
        ],
        out_specs=pl.BlockSpec((_BB, 1), lambda b: (b, 0)),
        out_shape=jax.ShapeDtypeStruct((_B, 1), f32),
    )(ni, nj, xi, xj, xt,
      xcn_w1, xcn_b1.reshape(1, 256), xcn_w2, xcn_b2.reshape(1, 256),
      xij_w1, xij_b1.reshape(1, 256), xij_w2, xij_b2.reshape(1, 256),
      lin0_w, lin0_b.reshape(1, 256), lin1_w, lin1_b.reshape(1, 256),
      lin2_w, lin2_b.reshape(1, 1))
    return jnp.squeeze(out, axis=1)


# speedup vs baseline: 1.0639x; 1.0639x over previous
import functools

import jax
import jax.numpy as jnp
from jax import lax
from jax.experimental import pallas as pl
from jax.experimental.pallas import tpu as pltpu
from jax.experimental.pallas import tpu_sc as plsc

_N = 10000
_B = 2048
_DIN = 128
_W = 1280
_PLANES = 8
_R = 4224

_NC, _NS = 2, 16
_NW = _NC * _NS
_BPW = _B // _NW

_BB = 256
_GRID = _B // _BB


@functools.cache
def _make_sc_gather():
    @functools.partial(
        pl.kernel,
        mesh=plsc.VectorSubcoreMesh(core_axis_name="c", subcore_axis_name="s"),
        out_type=(
            jax.ShapeDtypeStruct((_B, _W), jnp.int32),
            jax.ShapeDtypeStruct((_B, _W), jnp.int32),
            jax.ShapeDtypeStruct((_B, _DIN), jnp.float32),
            jax.ShapeDtypeStruct((_B, _DIN), jnp.float32),
        ),
        scratch_types=[
            pltpu.VMEM((_BPW,), jnp.int32),
            pltpu.VMEM((_BPW, _W), jnp.int32),
            pltpu.VMEM((_BPW, _DIN), jnp.float32),
            pltpu.SemaphoreType.DMA,
        ],
    )
    def _sc_gather(nib_hbm, x_hbm, ci_hbm, cj_hbm, ii_hbm, jj_hbm,
                   ni_out, nj_out, xi_out, xj_out,
                   idx_v, rowsn_v, rowsx_v, sem):
        wid = lax.axis_index("s") * _NC + lax.axis_index("c")
        base = wid * _BPW
        pltpu.sync_copy(ci_hbm.at[pl.ds(base, _BPW)], idx_v)
        pltpu.async_copy(nib_hbm.at[idx_v], rowsn_v, sem).wait()
        pltpu.sync_copy(rowsn_v, ni_out.at[pl.ds(base, _BPW)])
        pltpu.sync_copy(ii_hbm.at[pl.ds(base, _BPW)], idx_v)
        pltpu.async_copy(x_hbm.at[idx_v], rowsx_v, sem).wait()
        pltpu.sync_copy(rowsx_v, xi_out.at[pl.ds(base, _BPW)])
        pltpu.sync_copy(cj_hbm.at[pl.ds(base, _BPW)], idx_v)
        pltpu.async_copy(nib_hbm.at[idx_v], rowsn_v, sem).wait()
        pltpu.sync_copy(rowsn_v, nj_out.at[pl.ds(base, _BPW)])
        pltpu.sync_copy(jj_hbm.at[pl.ds(base, _BPW)], idx_v)
        pltpu.async_copy(x_hbm.at[idx_v], rowsx_v, sem).wait()
        pltpu.sync_copy(rowsx_v, xj_out.at[pl.ds(base, _BPW)])

    return _sc_gather


def _tc_body(ni_ref, nj_ref, xi_ref, xj_ref, xt_ref,
             xcn_w1_ref, xcn_b1_ref, xcn_w2_ref, xcn_b2_ref,
             xij_w1_ref, xij_b1_ref, xij_w2_ref, xij_b2_ref,
             lin0_w_ref, lin0_b_ref, lin1_w_ref, lin1_b_ref,
             lin2_w_ref, lin2_b_ref, out_ref):
    f32 = jnp.float32
    ni = ni_ref[...]
    nj = nj_ref[...]
    acc = jnp.zeros((_BB, _DIN), f32)
    for p in range(_PLANES):
        s = 4 * p
        mi = (jnp.right_shift(ni, s) & 15) > 0
        mj = (jnp.right_shift(nj, s) & 15) > 0
        m = (mi & mj).astype(f32)
        acc = acc + jnp.dot(m, xt_ref[p], preferred_element_type=f32)
    relu = lambda a: jnp.maximum(a, 0.0)
    xcn = relu(jnp.dot(acc, xcn_w1_ref[...], preferred_element_type=f32) + xcn_b1_ref[...])
    xcn = jnp.dot(xcn, xcn_w2_ref[...], preferred_element_type=f32) + xcn_b2_ref[...]
    xij = xi_ref[...] * xj_ref[...]
    xij = relu(jnp.dot(xij, xij_w1_ref[...], preferred_element_type=f32) + xij_b1_ref[...])
    xij = jnp.dot(xij, xij_w2_ref[...], preferred_element_type=f32) + xij_b2_ref[...]
    h = xcn + xij
    h = relu(jnp.dot(h, lin0_w_ref[...], preferred_element_type=f32) + lin0_b_ref[...])
    h = relu(jnp.dot(h, lin1_w_ref[...], preferred_element_type=f32) + lin1_b_ref[...])
    out_ref[...] = jnp.dot(h, lin2_w_ref[...], preferred_element_type=f32) + lin2_b_ref[...]


def _full(shape):
    return pl.BlockSpec(shape, lambda b: tuple(0 for _ in shape))


def kernel(x, adj_t, tar_ei, xcn_w1, xcn_b1, xcn_w2, xcn_b2,
           xij_w1, xij_b1, xij_w2, xij_b2,
           lin0_w, lin0_b, lin1_w, lin1_b, lin2_w, lin2_b):
    f32 = jnp.float32
    x = x.astype(f32)

    u = adj_t[0].astype(jnp.int32)
    v = adj_t[1].astype(jnp.int32)
    ii = tar_ei[0].astype(jnp.int32)
    jj = tar_ei[1].astype(jnp.int32)

    mask = jnp.zeros((_N,), jnp.int32).at[ii].set(1).at[jj].set(1)
    rowid = jnp.cumsum(mask) - mask
    ci = jnp.take(rowid, ii)
    cj = jnp.take(rowid, jj)

    rel = jnp.take(mask, u) > 0
    e_idx = jnp.arange(u.shape[0], dtype=jnp.int32)
    spread = 4096 * _W + e_idx % ((_R - 4096) * _W)
    widx = jnp.where(rel, jnp.take(rowid, u) * _W + jnp.right_shift(v, 3), spread)
    nibval = jnp.where(rel, jnp.left_shift(jnp.int32(1), jnp.left_shift(v & 7, 2)), 0)
    nib = jnp.zeros((_R * _W,), jnp.int32).at[widx].add(nibval).reshape(_R, _W)

    xp = jnp.concatenate([x, jnp.zeros((_W * _PLANES - _N, _DIN), f32)], axis=0)
    xt = xp.reshape(_W, _PLANES, _DIN).transpose(1, 0, 2)

    ni, nj, xi, xj = _make_sc_gather()(nib, x, ci, cj, ii, jj)

    out = pl.pallas_call(
        _tc_body,
        grid=(_GRID,),
        in_specs=[
            pl.BlockSpec((_BB, _W), lambda b: (b, 0)),
            pl.BlockSpec((_BB, _W), lambda b: (b, 0)),
            pl.BlockSpec((_BB, _DIN), lambda b: (b, 0)),
            pl.BlockSpec((_BB, _DIN), lambda b: (b, 0)),
            _full((_PLANES, _W, _DIN)),
            _full((_DIN, 256)), _full((1, 256)),
            _full((256, 256)), _full((1, 256)),
            _full((_DIN, 256)), _full((1, 256)),
            _full((256, 256)), _full((1, 256)),
            _full((256, 256)), _full((1, 256)),
            _full((256, 256)), _full((1, 256)),
            _full((256, 1)), _full((1, 1)),
        ],
        out_specs=pl.BlockSpec((_BB, 1), lambda b: (b, 0)),
        out_shape=jax.ShapeDtypeStruct((_B, 1), f32),
    )(ni, nj, xi, xj, xt,
      xcn_w1, xcn_b1.reshape(1, 256), xcn_w2, xcn_b2.reshape(1, 256),
      xij_w1, xij_b1.reshape(1, 256), xij_w2, xij_b2.reshape(1, 256),
      lin0_w, lin0_b.reshape(1, 256), lin1_w, lin1_b.reshape(1, 256),
      lin2_w, lin2_b.reshape(1, 1))
    return jnp.squeeze(out, axis=1)
